# baseline (speedup 1.0000x reference)
import functools

import jax
import jax.numpy as jnp
import numpy as np
from jax import lax
from jax.experimental import pallas as pl
from jax.experimental.pallas import tpu as pltpu
from jax.experimental.pallas import tpu_sc as plsc

N = 10000
E = 160000
D = 256
H = 512
L = 5
G = 32
T = 12
HALF = 128
BM = 1000
GRID = N // BM
NC, NS = 2, 16
NP = N + 16
RW = 624
RT = N - NS * RW
ET = E + N
SR = 1408
SRE = SR * 128
JG = SR // NS // 8
MB = 1024
MG = SRE // MB


@functools.cache
def _mesh():
  return plsc.VectorSubcoreMesh(
      core_axis_name="c", subcore_axis_name="s", num_cores=NC, num_subcores=NS)


def _gather_sc(htab, srcadj2d):
  @functools.partial(
      pl.kernel,
      out_type=jax.ShapeDtypeStruct((2 * SRE, HALF), jnp.float32),
      mesh=_mesh(),
      scratch_types=[
          pltpu.VMEM((8, 128), jnp.int32),
          pltpu.VMEM((128, HALF), jnp.float32),
      ])
  def k(htab_hbm, src_hbm, out_hbm, sidx, rows):
    c = lax.axis_index("c")
    s = lax.axis_index("s")

    @pl.loop(0, JG)
    def _(jb):
      rb = s * (8 * JG) + jb * 8
      pltpu.sync_copy(src_hbm.at[pl.ds(c * SR + rb, 8)], sidx)
      for kk in range(8):
        pltpu.sync_copy(htab_hbm.at[sidx.at[kk]], rows)
        pltpu.sync_copy(rows, out_hbm.at[pl.ds((c * SR + rb + kk) * 128, 128)])

  return k(htab, srcadj2d)


def _fold_sc(msg, dst2d, starts, zeros_n):
  @functools.partial(
      pl.kernel,
      out_type=jax.ShapeDtypeStruct((2 * N, HALF), jnp.float32),
      mesh=_mesh(),
      scratch_types=[
          pltpu.VMEM((8, 128), jnp.int32),
          pltpu.VMEM((128, HALF), jnp.float32),
          pltpu.VMEM_SHARED((NP, HALF), jnp.float32),
          pltpu.VMEM((256,), jnp.int32),
      ])
  def k(msg_hbm, dst_hbm, st_hbm, z_hbm, out_hbm, didx, buf, shared, stv):
    c = lax.axis_index("c")
    s = lax.axis_index("s")
    pltpu.sync_copy(z_hbm.at[pl.ds(s * RW, RW)], shared.at[pl.ds(s * RW, RW)])

    @pl.when(s == NS - 1)
    def _():
      pltpu.sync_copy(z_hbm.at[pl.ds(NS * RW, RT)],
                      shared.at[pl.ds(NS * RW, RT)])

    pltpu.sync_copy(st_hbm, stv)
    plsc.subcore_barrier()

    vv = stv[pl.ds(s * 16, 16)]
    start = vv[0]
    end = vv[1]
    b0 = start // 1024
    b1 = (end + 1023) // 1024
    iota = lax.broadcasted_iota(jnp.int32, (16,), 0)

    @pl.loop(b0, b1)
    def _(jb):
      rb = jb * 8
      pltpu.sync_copy(dst_hbm.at[pl.ds(rb, 8)], didx)
      for kk in range(8):
        r = rb + kk
        for j in range(8):
          p = r * 128 + j * 16 + iota
          ok = (p >= start) & (p < end)
          v = didx[kk, pl.ds(j * 16, 16)]
          didx[kk, pl.ds(j * 16, 16)] = jnp.where(ok, v, N)
        pltpu.sync_copy(msg_hbm.at[pl.ds((c * SR + r) * 128, 128)], buf)
        pltpu.sync_copy(buf, shared.at[didx.at[kk]], add=True)

    plsc.subcore_barrier()
    pltpu.sync_copy(shared.at[pl.ds(s * RW, RW)],
                    out_hbm.at[pl.ds(c * N + s * RW, RW)])

    @pl.when(s == NS - 1)
    def _():
      pltpu.sync_copy(shared.at[pl.ds(NS * RW, RT)],
                      out_hbm.at[pl.ds(c * N + NS * RW, RT)])

  return k(msg, dst2d, starts, zeros_n)


def _embed_tc(x0r, x1r, wt, wc):
  def body(x0_ref, x1_ref, wt_ref, wc_ref, o_ref):
    x0 = x0_ref[0]
    x1 = x1_ref[0]
    oh0 = (lax.broadcasted_iota(jnp.int32, (128, BM), 0) == x0).astype(jnp.float32)
    oh1 = (lax.broadcasted_iota(jnp.int32, (8, BM), 0) == x1).astype(jnp.float32)
    h = lax.dot_general(oh0, wt_ref[...], (((0,), (0,)), ((), ())),
                        preferred_element_type=jnp.float32,
                        precision=lax.Precision.HIGHEST)
    h = h + lax.dot_general(oh1, wc_ref[...], (((0,), (0,)), ((), ())),
                            preferred_element_type=jnp.float32,
                            precision=lax.Precision.HIGHEST)
    o_ref[0] = h[:, :HALF]
    o_ref[1] = h[:, HALF:]

  return pl.pallas_call(
      body,
      grid=(GRID,),
      in_specs=[
          pl.BlockSpec((1, 1, BM), lambda i: (i, 0, 0)),
          pl.BlockSpec((1, 1, BM), lambda i: (i, 0, 0)),
          pl.BlockSpec((128, D), lambda i: (0, 0)),
          pl.BlockSpec((8, D), lambda i: (0, 0)),
      ],
      out_specs=pl.BlockSpec((2, BM, HALF), lambda i: (0, i, 0)),
      out_shape=jax.ShapeDtypeStruct((2, N, HALF), jnp.float32),
  )(x0r, x1r, wt, wc)


def _mkmsg_tc(gath, a0r, a1r, e1p, e2p):
  def body(g_ref, a0_ref, a1_ref, e1_ref, e2_ref, o_ref):
    a0 = a0_ref[0]
    a1 = a1_ref[0]
    oh0 = (lax.broadcasted_iota(jnp.int32, (8, MB), 0) == a0).astype(jnp.float32)
    oh1 = (lax.broadcasted_iota(jnp.int32, (8, MB), 0) == a1).astype(jnp.float32)
    e1row = lax.dot_general(oh0, e1_ref[...], (((0,), (0,)), ((), ())),
                            preferred_element_type=jnp.float32,
                            precision=lax.Precision.HIGHEST)
    e2row = lax.dot_general(oh1, e2_ref[...], (((0,), (0,)), ((), ())),
                            preferred_element_type=jnp.float32,
                            precision=lax.Precision.HIGHEST)
    eemb = e1row + e2row
    o_ref[0] = g_ref[0] + eemb[:, :HALF]
    o_ref[1] = g_ref[1] + eemb[:, HALF:]

  return pl.pallas_call(
      body,
      grid=(MG,),
      in_specs=[
          pl.BlockSpec((2, MB, HALF), lambda i: (0, i, 0)),
          pl.BlockSpec((1, 1, MB), lambda i: (i, 0, 0)),
          pl.BlockSpec((1, 1, MB), lambda i: (i, 0, 0)),
          pl.BlockSpec((8, D), lambda i: (0, 0)),
          pl.BlockSpec((8, D), lambda i: (0, 0)),
      ],
      out_specs=pl.BlockSpec((2, MB, HALF), lambda i: (0, i, 0)),
      out_shape=jax.ShapeDtypeStruct((2, SRE, HALF), jnp.float32),
  )(gath, a0r, a1r, e1p, e2p)


def _bn_terms(st_ref, var_ref, g_ref, be_ref):
  mean = st_ref[0:1, :] / float(N)
  inv = 1.0 / jnp.sqrt(var_ref[0:1, :] + 1e-5)
  return mean, inv, g_ref[0:1, :], be_ref[0:1, :]


def _prep_tc(h2raw, stats, var, g_r, be_r):
  def body(h_ref, st_ref, v_ref, g_ref, be_ref, hn_ref):
    mean, inv, g, be = _bn_terms(st_ref, v_ref, g_ref, be_ref)
    for c in range(2):
      sl = slice(c * HALF, (c + 1) * HALF)
      hn = (h_ref[c] - mean[:, sl]) * inv[:, sl] * g[:, sl] + be[:, sl]
      hn_ref[c] = jnp.maximum(hn, 0.0)

  return pl.pallas_call(
      body,
      grid=(GRID,),
      in_specs=[
          pl.BlockSpec((2, BM, HALF), lambda i: (0, i, 0)),
          pl.BlockSpec((8, D), lambda i: (0, 0)),
          pl.BlockSpec((8, D), lambda i: (0, 0)),
          pl.BlockSpec((8, D), lambda i: (0, 0)),
          pl.BlockSpec((8, D), lambda i: (0, 0)),
      ],
      out_specs=pl.BlockSpec((2, BM, HALF), lambda i: (0, i, 0)),
      out_shape=jax.ShapeDtypeStruct((2, N, HALF), jnp.float32),
  )(h2raw, stats, var, g_r, be_r)


def _mlp_tc(agg, w1l, b1r, w2l, b2r):
  def body(a_ref, w1_ref, b1_ref, w2_ref, b2_ref, h2_ref, st_ref, acc):
    i = pl.program_id(0)
    b16 = jnp.bfloat16
    a = jnp.concatenate([a_ref[0], a_ref[1]], axis=1).astype(b16)
    mid = jnp.dot(a, w1_ref[...].astype(b16), preferred_element_type=jnp.float32)
    mid = jnp.maximum(mid + b1_ref[0:1, :], 0.0)
    h2 = jnp.dot(mid.astype(b16), w2_ref[...].astype(b16),
                 preferred_element_type=jnp.float32)
    h2 = h2 + b2_ref[0:1, :]
    h2_ref[0] = h2[:, :HALF]
    h2_ref[1] = h2[:, HALF:]

    @pl.when(i == 0)
    def _():
      acc[...] = jnp.zeros_like(acc)

    acc[0:1, :] += jnp.sum(h2, axis=0, keepdims=True)
    st_ref[...] = acc[...]

  return pl.pallas_call(
      body,
      grid=(GRID,),
      in_specs=[
          pl.BlockSpec((2, BM, HALF), lambda i: (0, i, 0)),
          pl.BlockSpec((D, H), lambda i: (0, 0)),
          pl.BlockSpec((8, H), lambda i: (0, 0)),
          pl.BlockSpec((H, D), lambda i: (0, 0)),
          pl.BlockSpec((8, D), lambda i: (0, 0)),
      ],
      out_specs=[
          pl.BlockSpec((2, BM, HALF), lambda i: (0, i, 0)),
          pl.BlockSpec((8, D), lambda i: (0, 0)),
      ],
      out_shape=[
          jax.ShapeDtypeStruct((2, N, HALF), jnp.float32),
          jax.ShapeDtypeStruct((8, D), jnp.float32),
      ],
      scratch_shapes=[pltpu.VMEM((8, D), jnp.float32)],
  )(agg, w1l, b1r, w2l, b2r)


def _var_tc(h2raw, stats):
  def body(h_ref, st_ref, v_ref, acc):
    i = pl.program_id(0)
    mean = st_ref[0:1, :] / float(N)
    h2 = jnp.concatenate([h_ref[0], h_ref[1]], axis=1)
    d = h2 - mean

    @pl.when(i == 0)
    def _():
      acc[...] = jnp.zeros_like(acc)

    acc[0:1, :] += jnp.sum(d * d, axis=0, keepdims=True)
    v_ref[...] = acc[...] / float(N)

  return pl.pallas_call(
      body,
      grid=(GRID,),
      in_specs=[
          pl.BlockSpec((2, BM, HALF), lambda i: (0, i, 0)),
          pl.BlockSpec((8, D), lambda i: (0, 0)),
      ],
      out_specs=pl.BlockSpec((8, D), lambda i: (0, 0)),
      out_shape=jax.ShapeDtypeStruct((8, D), jnp.float32),
      scratch_shapes=[pltpu.VMEM((8, D), jnp.float32)],
  )(h2raw, stats)


def _final_tc(h2raw, stats, var, g_r, be_r, batch3, pw_pad, pb_pad):
  def body(h_ref, st_ref, v_ref, g_ref, be_ref, b_ref, pw_ref, pb_ref, o_ref,
           gacc, cacc):
    i = pl.program_id(0)
    mean, inv, g, be = _bn_terms(st_ref, v_ref, g_ref, be_ref)
    hfull = jnp.concatenate([h_ref[0], h_ref[1]], axis=1)
    hfull = (hfull - mean) * inv * g + be
    bb = b_ref[0]
    oh = (lax.broadcasted_iota(jnp.int32, (G, BM), 0) == bb).astype(jnp.float32)

    @pl.when(i == 0)
    def _():
      gacc[...] = jnp.zeros_like(gacc)
      cacc[...] = jnp.zeros_like(cacc)

    gacc[...] += jnp.dot(oh, hfull, preferred_element_type=jnp.float32,
                         precision=lax.Precision.HIGHEST)
    cacc[...] += jnp.dot(oh, jnp.ones((BM, 8), jnp.float32),
                         preferred_element_type=jnp.float32,
                         precision=lax.Precision.HIGHEST)
    rep = gacc[...] / jnp.maximum(cacc[:, 0:1], 1.0)
    o_ref[...] = jnp.dot(rep.astype(jnp.bfloat16),
                         pw_ref[...].astype(jnp.bfloat16),
                         preferred_element_type=jnp.float32) + pb_ref[0:1, :]

  return pl.pallas_call(
      body,
      grid=(GRID,),
      in_specs=[
          pl.BlockSpec((2, BM, HALF), lambda i: (0, i, 0)),
          pl.BlockSpec((8, D), lambda i: (0, 0)),
          pl.BlockSpec((8, D), lambda i: (0, 0)),
          pl.BlockSpec((8, D), lambda i: (0, 0)),
          pl.BlockSpec((8, D), lambda i: (0, 0)),
          pl.BlockSpec((1, 1, BM), lambda i: (i, 0, 0)),
          pl.BlockSpec((D, 128), lambda i: (0, 0)),
          pl.BlockSpec((8, 128), lambda i: (0, 0)),
      ],
      out_specs=pl.BlockSpec((G, 128), lambda i: (0, 0)),
      out_shape=jax.ShapeDtypeStruct((G, 128), jnp.float32),
      scratch_shapes=[
          pltpu.VMEM((G, D), jnp.float32),
          pltpu.VMEM((G, 8), jnp.float32),
      ],
  )(h2raw, stats, var, g_r, be_r, batch3, pw_pad, pb_pad)


def _row_pad(v, rows=8):
  return jnp.pad(v.reshape(1, -1), ((0, rows - 1), (0, 0)))


def kernel(x, edge_index, edge_attr, batch, W_type, W_chir, edge_emb1,
           edge_emb2, w1, b1, w2, b2, gamma, beta, pred_w, pred_b):
  xi = x.astype(jnp.int32)
  ei = edge_index.astype(jnp.int32)
  ea = edge_attr.astype(jnp.int32)
  bt = batch.astype(jnp.int32)

  x0r = xi[:, 0].reshape(GRID, 1, BM)
  x1r = xi[:, 1].reshape(GRID, 1, BM)
  batch3 = bt.reshape(GRID, 1, BM)

  loop = jnp.arange(N, dtype=jnp.int32)
  src_ext = jnp.concatenate([ei[0], loop])
  dst_ext = jnp.concatenate([ei[1], loop])
  a0_ext = jnp.concatenate([ea[:, 0], jnp.full((N,), 4, jnp.int32)])
  a1_ext = jnp.concatenate([ea[:, 1], jnp.zeros((N,), jnp.int32)])
  perm = jnp.argsort(dst_ext, stable=True)
  pad2 = SRE - ET
  dstS = jnp.concatenate([dst_ext[perm], jnp.full((pad2,), N, jnp.int32)])
  srcS = jnp.concatenate([src_ext[perm], jnp.zeros((pad2,), jnp.int32)])
  a0S = jnp.concatenate([a0_ext[perm], jnp.full((pad2,), 7, jnp.int32)])
  a1S = jnp.concatenate([a1_ext[perm], jnp.full((pad2,), 7, jnp.int32)])

  srcadj2d = jnp.concatenate([srcS, srcS + N]).reshape(2 * SR, 128)
  dst2d = dstS.reshape(SR, 128)
  a0r = a0S.reshape(MG, 1, MB)
  a1r = a1S.reshape(MG, 1, MB)

  W = SRE // NS
  pos = jnp.arange(1, NS) * W
  vals = dstS[pos]
  right = jnp.searchsorted(dstS, vals, side="right").astype(jnp.int32)
  new_row = dstS[pos - 1] != vals
  starts_mid = jnp.where(new_row, pos.astype(jnp.int32), right)
  starts17 = jnp.concatenate([
      jnp.zeros((1,), jnp.int32), starts_mid, jnp.full((1,), SRE, jnp.int32)])
  starts = jnp.pad(
      jnp.stack([starts17[:NS], starts17[1:NS + 1]], axis=1),
      ((0, 0), (0, 14))).reshape(256)

  wt = jnp.pad(W_type, ((0, 128 - W_type.shape[0]), (0, 0)))
  wc = jnp.pad(W_chir, ((0, 8 - W_chir.shape[0]), (0, 0)))
  zeros_n = jnp.zeros((N, 128), jnp.float32)

  h0 = _embed_tc(x0r, x1r, wt, wc)
  e1ps = [jnp.pad(edge_emb1[l], ((0, 2), (0, 0))) for l in range(L)]
  e2ps = [jnp.pad(edge_emb2[l], ((0, 5), (0, 0))) for l in range(L)]

  h2raw, stats, var = None, None, None
  for l in range(L):
    if l == 0:
      tab = h0.reshape(2 * N, HALF)
    else:
      hn = _prep_tc(h2raw, stats, var, _row_pad(gamma[l - 1]),
                    _row_pad(beta[l - 1]))
      tab = hn.reshape(2 * N, HALF)
    gath = _gather_sc(tab, srcadj2d).reshape(2, SRE, HALF)
    msg = _mkmsg_tc(gath, a0r, a1r, e1ps[l], e2ps[l])
    agg = _fold_sc(msg.reshape(2 * SRE, HALF), dst2d, starts, zeros_n)
    h2raw, stats = _mlp_tc(agg.reshape(2, N, HALF), w1[l], _row_pad(b1[l]),
                           w2[l], _row_pad(b2[l]))
    var = _var_tc(h2raw, stats)

  pw_pad = jnp.pad(pred_w, ((0, 0), (0, 128 - T)))
  pb_pad = jnp.pad(pred_b.reshape(1, T), ((0, 7), (0, 128 - T)))
  out = _final_tc(h2raw, stats, var, _row_pad(gamma[L - 1]),
                  _row_pad(beta[L - 1]), batch3, pw_pad, pb_pad)
  return out[:, :T]
